# baseline (speedup 1.0000x reference)
import dataclasses
import functools

import jax
import jax.numpy as jnp
from jax.experimental import pallas as pl
from jax.experimental.pallas import tpu as pltpu
from jax.experimental.pallas import tpu_sc as plsc

_SEG = 8192
_V = 16
_U = 8

_TC_ROWS = 320
_TC_BLOCK_ROWS = 64


def _tc_body(x_ref, o_ref):
    e = jnp.exp(x_ref[...])
    s = jnp.sum(e, axis=1, keepdims=True)
    o_ref[...] = e * (1.0 / s)


def _tc_softmax(x):
    rows = x.shape[0]
    return pl.pallas_call(
        _tc_body,
        grid=(rows // _TC_BLOCK_ROWS,),
        in_specs=[pl.BlockSpec((_TC_BLOCK_ROWS, _SEG), lambda i: (i, 0))],
        out_specs=pl.BlockSpec((_TC_BLOCK_ROWS, _SEG), lambda i: (i, 0)),
        out_shape=jax.ShapeDtypeStruct((rows, _SEG), x.dtype),
    )(x)


def _sc_row_body(x_vmem, o_vmem):
    zeros = tuple(jnp.zeros((_V,), jnp.float32) for _ in range(_U))

    def p1(i, carry):
        out = []
        for u in range(_U):
            e = jnp.exp(x_vmem[0, pl.ds(i + u * _V, _V)])
            o_vmem[0, pl.ds(i + u * _V, _V)] = e
            out.append(carry[u] + e)
        return tuple(out)

    accs = plsc.parallel_loop(0, _SEG, step=_U * _V, unroll=4, carry=zeros)(p1)
    acc = sum(accs[1:], accs[0])
    r = jnp.ones((_V,), jnp.float32) / jnp.broadcast_to(jnp.sum(acc), (_V,))

    def p2(i):
        for u in range(_U):
            o_vmem[0, pl.ds(i + u * _V, _V)] = (
                o_vmem[0, pl.ds(i + u * _V, _V)] * r)

    plsc.parallel_loop(0, _SEG, step=_U * _V, unroll=4)(p2)


def _sc_softmax(x):
    rows = x.shape[0]
    mesh = plsc.VectorSubcoreMesh(core_axis_name="c", subcore_axis_name="s")

    @functools.partial(
        pl.kernel,
        out_type=jax.ShapeDtypeStruct((rows, _SEG), x.dtype),
        mesh=mesh,
        compiler_params=dataclasses.replace(
            pltpu.CompilerParams(), needs_layout_passes=False),
    )
    def sc_fn(x_hbm, o_hbm):
        pltpu.emit_pipeline(
            _sc_row_body,
            grid=(rows,),
            in_specs=[pl.BlockSpec((1, _SEG), lambda i: (i, 0))],
            out_specs=[pl.BlockSpec((1, _SEG), lambda i: (i, 0))],
            core_axis_name=("c", "s"),
            dimension_semantics=(pltpu.PARALLEL,),
        )(x_hbm, o_hbm)

    return sc_fn(x)


def kernel(tensor, reduce_indices):
    del reduce_indices
    b, total = tensor.shape
    rows = b * (total // _SEG)
    x = tensor.reshape(rows, _SEG)
    tc_out = _tc_softmax(x[:_TC_ROWS])
    sc_out = _sc_softmax(x[_TC_ROWS:])
    return jnp.concatenate([tc_out, sc_out], axis=0).reshape(b, total)


# speedup vs baseline: 1.8617x; 1.8617x over previous
import functools

import jax
import jax.numpy as jnp
from jax.experimental import pallas as pl
from jax.experimental.pallas import tpu as pltpu

_SEG = 8192
_ROWS_PER_BLOCK = 64


def _tc_block_body(x_vmem, o_vmem):
    e = jnp.exp(x_vmem[...])
    s = jnp.sum(e, axis=1, keepdims=True)
    o_vmem[...] = e * (1.0 / s)


def kernel(tensor, reduce_indices):
    del reduce_indices
    b, total = tensor.shape
    rows = b * (total // _SEG)
    x = tensor.reshape(rows, _SEG)
    mesh = pltpu.create_tensorcore_mesh("core")

    @functools.partial(
        pl.kernel,
        out_type=jax.ShapeDtypeStruct((rows, _SEG), tensor.dtype),
        mesh=mesh,
    )
    def tc_fn(x_hbm, o_hbm):
        pltpu.emit_pipeline(
            _tc_block_body,
            grid=(rows // _ROWS_PER_BLOCK,),
            in_specs=[pl.BlockSpec((_ROWS_PER_BLOCK, _SEG), lambda i: (i, 0))],
            out_specs=[pl.BlockSpec((_ROWS_PER_BLOCK, _SEG),
                                    lambda i: (i, 0))],
            core_axis_name="core",
            dimension_semantics=(pltpu.PARALLEL,),
        )(x_hbm, o_hbm)

    return tc_fn(x).reshape(b, total)


# speedup vs baseline: 7.1419x; 3.8362x over previous
import jax
import jax.numpy as jnp
from jax.experimental import pallas as pl

_SEG = 8192
_SEGS_PER_BLOCK = 8


def _tc_body(x_ref, o_ref):
    for j in range(_SEGS_PER_BLOCK):
        sl = (slice(None), slice(j * _SEG, (j + 1) * _SEG))
        e = jnp.exp(x_ref[sl])
        s = jnp.sum(e, axis=1, keepdims=True)
        o_ref[sl] = e * (1.0 / s)


def kernel(tensor, reduce_indices):
    del reduce_indices
    b, total = tensor.shape
    nblk = total // (_SEG * _SEGS_PER_BLOCK)
    return pl.pallas_call(
        _tc_body,
        grid=(nblk,),
        in_specs=[pl.BlockSpec((b, _SEG * _SEGS_PER_BLOCK), lambda i: (0, i))],
        out_specs=pl.BlockSpec((b, _SEG * _SEGS_PER_BLOCK), lambda i: (0, i)),
        out_shape=jax.ShapeDtypeStruct((b, total), tensor.dtype),
    )(tensor)


# speedup vs baseline: 7.7072x; 1.0791x over previous
import jax
import jax.numpy as jnp
from jax.experimental import pallas as pl

_SEG = 8192
_SEGS_PER_BLOCK = 16


def _tc_body(x_ref, o_ref):
    for j in range(_SEGS_PER_BLOCK):
        sl = (slice(None), slice(j * _SEG, (j + 1) * _SEG))
        e = jnp.exp(x_ref[sl])
        s = jnp.sum(e, axis=1, keepdims=True)
        o_ref[sl] = e * (1.0 / s)


def kernel(tensor, reduce_indices):
    del reduce_indices
    b, total = tensor.shape
    nblk = total // (_SEG * _SEGS_PER_BLOCK)
    return pl.pallas_call(
        _tc_body,
        grid=(nblk,),
        in_specs=[pl.BlockSpec((b, _SEG * _SEGS_PER_BLOCK), lambda i: (0, i))],
        out_specs=pl.BlockSpec((b, _SEG * _SEGS_PER_BLOCK), lambda i: (0, i)),
        out_shape=jax.ShapeDtypeStruct((b, total), tensor.dtype),
    )(tensor)


# speedup vs baseline: 8.8647x; 1.1502x over previous
import jax
import jax.numpy as jnp
from jax.experimental import pallas as pl

_SEG = 8192
_SEGS_PER_BLOCK = 32


def _tc_body(x_ref, o_ref):
    for j in range(_SEGS_PER_BLOCK):
        sl = (slice(None), slice(j * _SEG, (j + 1) * _SEG))
        e = jnp.exp(x_ref[sl])
        s = jnp.sum(e, axis=1, keepdims=True)
        o_ref[sl] = e * (1.0 / s)


def kernel(tensor, reduce_indices):
    del reduce_indices
    b, total = tensor.shape
    nblk = total // (_SEG * _SEGS_PER_BLOCK)
    return pl.pallas_call(
        _tc_body,
        grid=(nblk,),
        in_specs=[pl.BlockSpec((b, _SEG * _SEGS_PER_BLOCK), lambda i: (0, i))],
        out_specs=pl.BlockSpec((b, _SEG * _SEGS_PER_BLOCK), lambda i: (0, i)),
        out_shape=jax.ShapeDtypeStruct((b, total), tensor.dtype),
    )(tensor)
